# Initial kernel scaffold; baseline (speedup 1.0000x reference)
import jax
import jax.numpy as jnp
from jax.experimental import pallas as pl


def kernel(x, edge_index, batch, W1, b1, W2, b2):
    raise NotImplementedError("write your pallas kernel here")


# speedup vs baseline: 10.6581x; 10.6581x over previous
import functools

import jax
import jax.numpy as jnp
from jax import lax
from jax.experimental import pallas as pl
from jax.experimental.pallas import tpu as pltpu
from jax.experimental.pallas import tpu_sc as plsc

N2 = 10240
E2 = 163840
D = 256
HD = 128
G = 64
RB = 256
NBLK = N2 // RB
CHUNK = 128
EPT = E2 // 16
NCH = EPT // CHUNK
EPT_D = E2 // 32
NCH_D = EPT_D // CHUNK
RPT = N2 // 16

_sc_mesh = plsc.VectorSubcoreMesh(core_axis_name="c", subcore_axis_name="s")


@functools.partial(
    pl.kernel,
    out_type=jax.ShapeDtypeStruct((2 * N2, HD), jnp.float32),
    mesh=_sc_mesh,
    scratch_types=[
        pltpu.VMEM((CHUNK,), jnp.int32),
        pltpu.VMEM((CHUNK, HD), jnp.float32),
        pltpu.VMEM_SHARED((N2, HD), jnp.float32),
        pltpu.SemaphoreType.DMA,
    ],
)
def _deg_kernel(dst_hbm, zeros_hbm, ones_hbm, out_hbm, didx, ones_v, acc, sem):
    c = lax.axis_index("c")
    s = lax.axis_index("s")
    pltpu.sync_copy(ones_hbm, ones_v)
    pltpu.sync_copy(zeros_hbm, acc.at[pl.ds(s * RPT, RPT)])
    plsc.subcore_barrier()
    base = c * (E2 // 2) + s * EPT_D

    def body(i, carry):
        eb = pl.multiple_of(base + i * CHUNK, 8)
        pltpu.sync_copy(dst_hbm.at[pl.ds(eb, CHUNK)], didx)
        pltpu.sync_copy(ones_v, acc.at[didx], add=True)
        return carry

    lax.fori_loop(0, NCH_D, body, 0)
    plsc.subcore_barrier()
    pltpu.sync_copy(
        acc.at[pl.ds(s * RPT, RPT)],
        out_hbm.at[pl.ds(c * N2 + s * RPT, RPT)],
    )


@functools.partial(
    pl.kernel,
    out_type=jax.ShapeDtypeStruct((2 * N2, HD), jnp.float32),
    mesh=_sc_mesh,
    scratch_types=[
        pltpu.VMEM((CHUNK,), jnp.int32),
        pltpu.VMEM((CHUNK,), jnp.int32),
        pltpu.VMEM((CHUNK,), jnp.int32),
        pltpu.VMEM((CHUNK, HD), jnp.float32),
        pltpu.VMEM((CHUNK,), jnp.int32),
        pltpu.VMEM((CHUNK,), jnp.int32),
        pltpu.VMEM((CHUNK,), jnp.int32),
        pltpu.VMEM((CHUNK, HD), jnp.float32),
        pltpu.VMEM_SHARED((N2, HD), jnp.float32),
        pltpu.SemaphoreType.DMA,
        pltpu.SemaphoreType.DMA,
    ],
)
def _agg_kernel(hs_hbm, src_hbm, dst_hbm, zeros_hbm, out_hbm,
                sa, da, ga, ra, sb, db, gb, rb, acc, sem_a, sem_b):
    c = lax.axis_index("c")
    s = lax.axis_index("s")
    pltpu.sync_copy(zeros_hbm, acc.at[pl.ds(s * RPT, RPT)])
    plsc.subcore_barrier()
    base = s * EPT

    def load_and_fire(eb, sidx, didx, gidx, rows, sem):
        pltpu.sync_copy(src_hbm.at[pl.ds(eb, CHUNK)], sidx)
        pltpu.sync_copy(dst_hbm.at[pl.ds(eb, CHUNK)], didx)
        for j in range(CHUNK // 16):
            sv = sidx[pl.ds(j * 16, 16)]
            gidx[pl.ds(j * 16, 16)] = sv * 2 + c
        return pltpu.async_copy(hs_hbm.at[gidx], rows, sem)

    def body(i, carry):
        eb_a = pl.multiple_of(base + (2 * i) * CHUNK, 8)
        eb_b = pl.multiple_of(base + (2 * i + 1) * CHUNK, 8)
        cp_a = load_and_fire(eb_a, sa, da, ga, ra, sem_a)
        cp_b = load_and_fire(eb_b, sb, db, gb, rb, sem_b)
        cp_a.wait()
        pltpu.sync_copy(ra, acc.at[da], add=True)
        cp_b.wait()
        pltpu.sync_copy(rb, acc.at[db], add=True)
        return carry

    lax.fori_loop(0, NCH // 2, body, 0)
    plsc.subcore_barrier()
    pltpu.sync_copy(acc.at[pl.ds(s * RPT, RPT)],
                    out_hbm.at[pl.ds(c * N2 + s * RPT, RPT)])


def _k1_body(x_ref, ca_ref, cb_ref, w_ref, o_ref):
    deg = 1.0 + ca_ref[:, 0:1] + cb_ref[:, 0:1]
    dinv = lax.rsqrt(deg)
    xs = x_ref[...] * dinv
    h = jnp.dot(xs, w_ref[...], preferred_element_type=jnp.float32)
    o_ref[:, 0, :] = h[:, :HD]
    o_ref[:, 1, :] = h[:, HD:]


def _k2_body(sa_ref, sb_ref, hs_ref, ca_ref, cb_ref, w_ref, b_ref, o_ref):
    deg = 1.0 + ca_ref[:, 0:1] + cb_ref[:, 0:1]
    dinv = lax.rsqrt(deg)
    agg = jnp.concatenate([sa_ref[...], sb_ref[...]], axis=1)
    hs = jnp.concatenate([hs_ref[:, 0, :], hs_ref[:, 1, :]], axis=1)
    t = jnp.maximum(dinv * (agg + hs) + b_ref[...], 0.0)
    h2 = jnp.dot(dinv * t, w_ref[...], preferred_element_type=jnp.float32)
    o_ref[:, 0, :] = h2[:, :HD]
    o_ref[:, 1, :] = h2[:, HD:]


def _k3_body(sa_ref, sb_ref, hs_ref, ca_ref, cb_ref, b_ref, batch_ref, o_ref):
    i = pl.program_id(0)
    deg = 1.0 + ca_ref[:, 0:1] + cb_ref[:, 0:1]
    dinv = lax.rsqrt(deg)
    agg = jnp.concatenate([sa_ref[...], sb_ref[...]], axis=1)
    hs = jnp.concatenate([hs_ref[:, 0, :], hs_ref[:, 1, :]], axis=1)
    out2 = dinv * (agg + hs) + b_ref[...]
    brow = batch_ref[0]
    gi = lax.broadcasted_iota(jnp.int32, (G, RB), 0)
    onehot = jnp.where(
        lax.broadcast_in_dim(brow, (G, RB), (0, 1)) == gi, 1.0, 0.0)
    contrib = jnp.dot(onehot, out2, preferred_element_type=jnp.float32)

    @pl.when(i == 0)
    def _():
        o_ref[...] = jnp.zeros_like(o_ref)

    o_ref[...] += contrib


def kernel(x, edge_index, batch, W1, b1, W2, b2):
    n, d = x.shape
    e = edge_index.shape[1]
    npad_n = N2 - n
    npad_e = E2 - e
    x_pad = jnp.concatenate([x, jnp.zeros((npad_n, d), x.dtype)], axis=0)
    ar = jnp.arange(npad_e, dtype=jnp.int32)
    src = jnp.concatenate([edge_index[0], ar % n])
    dst = jnp.concatenate([edge_index[1], n + (ar % npad_n)])
    batch_pad = jnp.concatenate(
        [batch, jnp.full((npad_n,), G, jnp.int32)]).reshape(NBLK, 1, RB)
    zeros_hd = jnp.zeros((RPT, HD), jnp.float32)
    ones_hd = jnp.ones((CHUNK, HD), jnp.float32)

    cnt = _deg_kernel(dst, zeros_hd, ones_hd)
    cnt_a = cnt[:N2]
    cnt_b = cnt[N2:]

    hs1 = pl.pallas_call(
        _k1_body,
        grid=(NBLK,),
        in_specs=[
            pl.BlockSpec((RB, D), lambda i: (i, 0)),
            pl.BlockSpec((RB, HD), lambda i: (i, 0)),
            pl.BlockSpec((RB, HD), lambda i: (i, 0)),
            pl.BlockSpec((D, D), lambda i: (0, 0)),
        ],
        out_specs=pl.BlockSpec((RB, 2, HD), lambda i: (i, 0, 0)),
        out_shape=jax.ShapeDtypeStruct((N2, 2, HD), jnp.float32),
    )(x_pad, cnt_a, cnt_b, W1)

    s1 = _agg_kernel(hs1.reshape(2 * N2, HD), src, dst, zeros_hd)
    s1a = s1[:N2]
    s1b = s1[N2:]

    hs2 = pl.pallas_call(
        _k2_body,
        grid=(NBLK,),
        in_specs=[
            pl.BlockSpec((RB, HD), lambda i: (i, 0)),
            pl.BlockSpec((RB, HD), lambda i: (i, 0)),
            pl.BlockSpec((RB, 2, HD), lambda i: (i, 0, 0)),
            pl.BlockSpec((RB, HD), lambda i: (i, 0)),
            pl.BlockSpec((RB, HD), lambda i: (i, 0)),
            pl.BlockSpec((D, D), lambda i: (0, 0)),
            pl.BlockSpec((1, D), lambda i: (0, 0)),
        ],
        out_specs=pl.BlockSpec((RB, 2, HD), lambda i: (i, 0, 0)),
        out_shape=jax.ShapeDtypeStruct((N2, 2, HD), jnp.float32),
    )(s1a, s1b, hs1, cnt_a, cnt_b, W2, b1.reshape(1, D))

    s2 = _agg_kernel(hs2.reshape(2 * N2, HD), src, dst, zeros_hd)
    s2a = s2[:N2]
    s2b = s2[N2:]

    out = pl.pallas_call(
        _k3_body,
        grid=(NBLK,),
        in_specs=[
            pl.BlockSpec((RB, HD), lambda i: (i, 0)),
            pl.BlockSpec((RB, HD), lambda i: (i, 0)),
            pl.BlockSpec((RB, 2, HD), lambda i: (i, 0, 0)),
            pl.BlockSpec((RB, HD), lambda i: (i, 0)),
            pl.BlockSpec((RB, HD), lambda i: (i, 0)),
            pl.BlockSpec((1, D), lambda i: (0, 0)),
            pl.BlockSpec((1, 1, RB), lambda i: (i, 0, 0)),
        ],
        out_specs=pl.BlockSpec((G, D), lambda i: (0, 0)),
        out_shape=jax.ShapeDtypeStruct((G, D), jnp.float32),
    )(s2a, s2b, hs2, cnt_a, cnt_b, b2.reshape(1, D), batch_pad)

    return out


# speedup vs baseline: 10.9690x; 1.0292x over previous
import functools

import jax
import jax.numpy as jnp
from jax import lax
from jax.experimental import pallas as pl
from jax.experimental.pallas import tpu as pltpu
from jax.experimental.pallas import tpu_sc as plsc

N2 = 10240
E2 = 163840
D = 256
HD = 128
G = 64
RB = 256
NBLK = N2 // RB
CHUNK = 128
EPT = E2 // 16
NCH = EPT // CHUNK
EPT_D = E2 // 32
NCH_D = EPT_D // CHUNK
RPT = N2 // 16

_sc_mesh = plsc.VectorSubcoreMesh(core_axis_name="c", subcore_axis_name="s")


@functools.partial(
    pl.kernel,
    out_type=jax.ShapeDtypeStruct((2 * N2, HD), jnp.float32),
    mesh=_sc_mesh,
    scratch_types=[
        pltpu.VMEM((CHUNK,), jnp.int32),
        pltpu.VMEM((CHUNK, HD), jnp.float32),
        pltpu.VMEM_SHARED((N2, HD), jnp.float32),
        pltpu.SemaphoreType.DMA,
    ],
)
def _deg_kernel(dst_hbm, zeros_hbm, ones_hbm, out_hbm, didx, ones_v, acc, sem):
    c = lax.axis_index("c")
    s = lax.axis_index("s")
    pltpu.sync_copy(ones_hbm, ones_v)
    pltpu.sync_copy(zeros_hbm, acc.at[pl.ds(s * RPT, RPT)])
    plsc.subcore_barrier()
    base = c * (E2 // 2) + s * EPT_D

    def body(i, carry):
        eb = pl.multiple_of(base + i * CHUNK, 8)
        pltpu.sync_copy(dst_hbm.at[pl.ds(eb, CHUNK)], didx)
        pltpu.sync_copy(ones_v, acc.at[didx], add=True)
        return carry

    lax.fori_loop(0, NCH_D, body, 0)
    plsc.subcore_barrier()
    pltpu.sync_copy(
        acc.at[pl.ds(s * RPT, RPT)],
        out_hbm.at[pl.ds(c * N2 + s * RPT, RPT)],
    )


NB = 2
SG = NCH // (2 * NB)
CPT = EPT // CHUNK


@functools.partial(
    pl.kernel,
    out_type=jax.ShapeDtypeStruct((2 * N2, HD), jnp.float32),
    mesh=_sc_mesh,
    scratch_types=[
        [pltpu.VMEM((CHUNK,), jnp.int32) for _ in range(NB)],
        [pltpu.VMEM((CHUNK,), jnp.int32) for _ in range(NB)],
        [pltpu.VMEM((CHUNK, HD), jnp.float32) for _ in range(NB)],
        [pltpu.VMEM((NB, CHUNK), jnp.int32) for _ in range(2)],
        [pltpu.VMEM((NB, CHUNK), jnp.int32) for _ in range(2)],
        pltpu.VMEM_SHARED((N2, HD), jnp.float32),
        [pltpu.SemaphoreType.DMA for _ in range(NB)],
        [pltpu.SemaphoreType.DMA for _ in range(NB)],
    ],
)
def _agg_kernel(hs_hbm, srcm_hbm, dstm_hbm, zeros_hbm, out_hbm,
                gidx, didx, rows, sblk, dblk, acc, gsem, ssem):
    c = lax.axis_index("c")
    s = lax.axis_index("s")
    pltpu.sync_copy(zeros_hbm, acc.at[pl.ds(s * RPT, RPT)])
    plsc.subcore_barrier()
    base = s * CPT

    def load_blk(p, row0):
        pltpu.sync_copy(srcm_hbm.at[pl.ds(row0, NB)], sblk[p])
        pltpu.sync_copy(dstm_hbm.at[pl.ds(row0, NB)], dblk[p])

    def fire_gather(b, p):
        for j in range(CHUNK // 16):
            sv = sblk[p][b, pl.ds(j * 16, 16)]
            gidx[b][pl.ds(j * 16, 16)] = sv * 2 + c
            didx[b][pl.ds(j * 16, 16)] = dblk[p][b, pl.ds(j * 16, 16)]
        return pltpu.async_copy(hs_hbm.at[gidx[b]], rows[b], gsem[b])

    def fire_scatter(b):
        return pltpu.async_copy(rows[b], acc.at[didx[b]], ssem[b],
                                add=True)

    def body(t, carry):
        row0 = base + t * 2 * NB
        load_blk(0, row0)
        gd = [fire_gather(k, 0) for k in range(NB)]
        load_blk(1, row0 + NB)
        sd = []
        for k in range(NB):
            gd[k].wait()
            sd.append(fire_scatter(k))
        gd2 = []
        for k in range(NB):
            sd[k].wait()
            gd2.append(fire_gather(k, 1))
        sd2 = []
        for k in range(NB):
            gd2[k].wait()
            sd2.append(fire_scatter(k))
        for k in range(NB):
            sd2[k].wait()
        return carry

    lax.fori_loop(0, SG, body, 0)
    plsc.subcore_barrier()
    pltpu.sync_copy(acc.at[pl.ds(s * RPT, RPT)],
                    out_hbm.at[pl.ds(c * N2 + s * RPT, RPT)])


def _k1_body(x_ref, ca_ref, cb_ref, w_ref, o_ref):
    deg = 1.0 + ca_ref[:, 0:1] + cb_ref[:, 0:1]
    dinv = lax.rsqrt(deg)
    xs = x_ref[...] * dinv
    h = jnp.dot(xs, w_ref[...], preferred_element_type=jnp.float32)
    o_ref[:, 0, :] = h[:, :HD]
    o_ref[:, 1, :] = h[:, HD:]


def _k2_body(sa_ref, sb_ref, hs_ref, ca_ref, cb_ref, w_ref, b_ref, o_ref):
    deg = 1.0 + ca_ref[:, 0:1] + cb_ref[:, 0:1]
    dinv = lax.rsqrt(deg)
    agg = jnp.concatenate([sa_ref[...], sb_ref[...]], axis=1)
    hs = jnp.concatenate([hs_ref[:, 0, :], hs_ref[:, 1, :]], axis=1)
    t = jnp.maximum(dinv * (agg + hs) + b_ref[...], 0.0)
    h2 = jnp.dot(dinv * t, w_ref[...], preferred_element_type=jnp.float32)
    o_ref[:, 0, :] = h2[:, :HD]
    o_ref[:, 1, :] = h2[:, HD:]


def _k3_body(sa_ref, sb_ref, hs_ref, ca_ref, cb_ref, b_ref, batch_ref, o_ref):
    i = pl.program_id(0)
    deg = 1.0 + ca_ref[:, 0:1] + cb_ref[:, 0:1]
    dinv = lax.rsqrt(deg)
    agg = jnp.concatenate([sa_ref[...], sb_ref[...]], axis=1)
    hs = jnp.concatenate([hs_ref[:, 0, :], hs_ref[:, 1, :]], axis=1)
    out2 = dinv * (agg + hs) + b_ref[...]
    brow = batch_ref[0]
    gi = lax.broadcasted_iota(jnp.int32, (G, RB), 0)
    onehot = jnp.where(
        lax.broadcast_in_dim(brow, (G, RB), (0, 1)) == gi, 1.0, 0.0)
    contrib = jnp.dot(onehot, out2, preferred_element_type=jnp.float32)

    @pl.when(i == 0)
    def _():
        o_ref[...] = jnp.zeros_like(o_ref)

    o_ref[...] += contrib


def kernel(x, edge_index, batch, W1, b1, W2, b2):
    n, d = x.shape
    e = edge_index.shape[1]
    npad_n = N2 - n
    npad_e = E2 - e
    x_pad = jnp.concatenate([x, jnp.zeros((npad_n, d), x.dtype)], axis=0)
    ar = jnp.arange(npad_e, dtype=jnp.int32)
    src = jnp.concatenate([edge_index[0], ar % n])
    dst = jnp.concatenate([edge_index[1], n + (ar % npad_n)])
    srcm = src.reshape(E2 // CHUNK, CHUNK)
    dstm = dst.reshape(E2 // CHUNK, CHUNK)
    batch_pad = jnp.concatenate(
        [batch, jnp.full((npad_n,), G, jnp.int32)]).reshape(NBLK, 1, RB)
    zeros_hd = jnp.zeros((RPT, HD), jnp.float32)
    ones_hd = jnp.ones((CHUNK, HD), jnp.float32)

    cnt = _deg_kernel(dst, zeros_hd, ones_hd)
    cnt_a = cnt[:N2]
    cnt_b = cnt[N2:]

    hs1 = pl.pallas_call(
        _k1_body,
        grid=(NBLK,),
        in_specs=[
            pl.BlockSpec((RB, D), lambda i: (i, 0)),
            pl.BlockSpec((RB, HD), lambda i: (i, 0)),
            pl.BlockSpec((RB, HD), lambda i: (i, 0)),
            pl.BlockSpec((D, D), lambda i: (0, 0)),
        ],
        out_specs=pl.BlockSpec((RB, 2, HD), lambda i: (i, 0, 0)),
        out_shape=jax.ShapeDtypeStruct((N2, 2, HD), jnp.float32),
    )(x_pad, cnt_a, cnt_b, W1)

    s1 = _agg_kernel(hs1.reshape(2 * N2, HD), srcm, dstm, zeros_hd)
    s1a = s1[:N2]
    s1b = s1[N2:]

    hs2 = pl.pallas_call(
        _k2_body,
        grid=(NBLK,),
        in_specs=[
            pl.BlockSpec((RB, HD), lambda i: (i, 0)),
            pl.BlockSpec((RB, HD), lambda i: (i, 0)),
            pl.BlockSpec((RB, 2, HD), lambda i: (i, 0, 0)),
            pl.BlockSpec((RB, HD), lambda i: (i, 0)),
            pl.BlockSpec((RB, HD), lambda i: (i, 0)),
            pl.BlockSpec((D, D), lambda i: (0, 0)),
            pl.BlockSpec((1, D), lambda i: (0, 0)),
        ],
        out_specs=pl.BlockSpec((RB, 2, HD), lambda i: (i, 0, 0)),
        out_shape=jax.ShapeDtypeStruct((N2, 2, HD), jnp.float32),
    )(s1a, s1b, hs1, cnt_a, cnt_b, W2, b1.reshape(1, D))

    s2 = _agg_kernel(hs2.reshape(2 * N2, HD), srcm, dstm, zeros_hd)
    s2a = s2[:N2]
    s2b = s2[N2:]

    out = pl.pallas_call(
        _k3_body,
        grid=(NBLK,),
        in_specs=[
            pl.BlockSpec((RB, HD), lambda i: (i, 0)),
            pl.BlockSpec((RB, HD), lambda i: (i, 0)),
            pl.BlockSpec((RB, 2, HD), lambda i: (i, 0, 0)),
            pl.BlockSpec((RB, HD), lambda i: (i, 0)),
            pl.BlockSpec((RB, HD), lambda i: (i, 0)),
            pl.BlockSpec((1, D), lambda i: (0, 0)),
            pl.BlockSpec((1, 1, RB), lambda i: (i, 0, 0)),
        ],
        out_specs=pl.BlockSpec((G, D), lambda i: (0, 0)),
        out_shape=jax.ShapeDtypeStruct((G, D), jnp.float32),
    )(s2a, s2b, hs2, cnt_a, cnt_b, b2.reshape(1, D), batch_pad)

    return out
